# baseline (speedup 1.0000x reference)
import functools

import jax
import jax.numpy as jnp
from jax import lax
from jax.experimental import pallas as pl
from jax.experimental.pallas import tpu as pltpu, tpu_sc as plsc

NC, NS, L = 2, 16, 16
NW = NC * NS
TPB = NS // 2
CH = 16


def _sc_kernel(B, C, H, W, N, Nv, K):
  HW = H * W
  NCHUNK = N // CH
  NV2 = Nv * 2 + (-Nv * 2) % 8
  NPH = 2 * ((NCHUNK // TPB + 2) // 2)
  mesh = plsc.VectorSubcoreMesh(
      core_axis_name="c", subcore_axis_name="s", num_cores=NC, num_subcores=NS)

  @functools.partial(
      pl.kernel,
      out_type=jax.ShapeDtypeStruct((B * N, C), jnp.bfloat16),
      mesh=mesh,
      compiler_params=pltpu.CompilerParams(
          needs_layout_passes=False, use_tc_tiling_on_sc=False),
      scratch_types=[
          pltpu.VMEM((NV2,), jnp.float32),
          pltpu.VMEM((K * 3,), jnp.float32),
          pltpu.VMEM((CH * 3,), jnp.int32),
          pltpu.VMEM((CH * 3,), jnp.int32),
          pltpu.VMEM((5 * CH,), jnp.int32),
          pltpu.VMEM((5 * CH,), jnp.int32),
          pltpu.VMEM((4 * CH, C), jnp.bfloat16),
          pltpu.VMEM((4 * CH, C), jnp.bfloat16),
          pltpu.VMEM((CH, C), jnp.bfloat16),
          pltpu.VMEM((CH, C), jnp.bfloat16),
          pltpu.SemaphoreType.DMA,
          pltpu.SemaphoreType.DMA,
          pltpu.SemaphoreType.DMA,
          pltpu.SemaphoreType.DMA,
          pltpu.SemaphoreType.DMA,
          pltpu.SemaphoreType.DMA,
      ],
  )
  def k(fm_hbm, verts_hbm, parents_hbm, bary_hbm, out_hbm,
        verts_v, bary_v, pv0, pv1, wv0, wv1, rows0, rows1, outv0, outv1,
        psem0, psem1, rsem0, rsem1, osem0, osem1):
    wid = lax.axis_index("s") * NC + lax.axis_index("c")
    b = wid // TPB
    s = wid % TPB
    pv = (pv0, pv1)
    wv = (wv0, wv1)
    rows = (rows0, rows1)
    outv = (outv0, outv1)
    psem = (psem0, psem1)
    rsem = (rsem0, rsem1)
    osem = (osem0, osem1)

    pltpu.sync_copy(verts_hbm.at[pl.ds(b * NV2, NV2)], verts_v)
    pltpu.sync_copy(bary_hbm, bary_v)

    lane = lax.broadcasted_iota(jnp.int32, (L,), 0)
    lane3 = lane * 3
    fmbase = b * HW

    def chunk_of(j):
      return s + TPB * j

    def fire_parents(j, pj):
      c = chunk_of(j)
      pltpu.async_copy(
          parents_hbm.at[pl.ds(c * (CH * 3), CH * 3)], pv[pj], psem[pj])

    def wait_parents(pj):
      pltpu.make_async_copy(
          parents_hbm.at[pl.ds(0, CH * 3)], pv[pj], psem[pj]).wait()

    def compute_and_fire(j, pj):
      c = chunk_of(j)
      n0 = c * CH
      nvec = n0 + lane
      bidx = lax.rem(nvec, K) * 3
      b0 = plsc.load_gather(bary_v, [bidx])
      b1 = plsc.load_gather(bary_v, [bidx + 1])
      b2 = plsc.load_gather(bary_v, [bidx + 2])
      p0 = plsc.load_gather(pv[pj], [lane3]) * 2
      p1 = plsc.load_gather(pv[pj], [lane3 + 1]) * 2
      p2 = plsc.load_gather(pv[pj], [lane3 + 2]) * 2
      vx = (b0 * plsc.load_gather(verts_v, [p0])
            + b1 * plsc.load_gather(verts_v, [p1])
            + b2 * plsc.load_gather(verts_v, [p2]))
      vy = (b0 * plsc.load_gather(verts_v, [p0 + 1])
            + b1 * plsc.load_gather(verts_v, [p1 + 1])
            + b2 * plsc.load_gather(verts_v, [p2 + 1]))
      x = vx / (W - 1.0) * 2.0 - 1.0
      y = vy / (H - 1.0) * 2.0 - 1.0
      ix = (x + 1.0) * 0.5 * (W - 1.0)
      iy = (y + 1.0) * 0.5 * (H - 1.0)
      txi = ix.astype(jnp.int32)
      ix0 = jnp.where(ix < txi.astype(jnp.float32), txi - 1, txi)
      tyi = iy.astype(jnp.int32)
      iy0 = jnp.where(iy < tyi.astype(jnp.float32), tyi - 1, tyi)
      fx = ix - ix0.astype(jnp.float32)
      fy = iy - iy0.astype(jnp.float32)
      wx = (1.0 - fx, fx)
      wy = (1.0 - fy, fy)
      for ci, (dx, dy) in enumerate(((0, 0), (1, 0), (0, 1), (1, 1))):
        xi = ix0 + dx
        yi = iy0 + dy
        valid = ((xi >= 0) & (xi <= W - 1) & (yi >= 0) & (yi <= H - 1))
        xc = jnp.clip(xi, 0, W - 1)
        yc = jnp.clip(yi, 0, H - 1)
        lin = fmbase + yc * W + xc
        pltpu.async_copy(
            fm_hbm.at[lin], rows[pj].at[pl.ds(ci * CH, CH)], rsem[pj])
        w = jnp.where(valid, wx[dx] * wy[dy], 0.0)
        wpair = plsc.bitcast(
            plsc.pack(w, w, format=plsc.PackFormat.INTERLEAVED), jnp.int32)
        wv[pj][pl.ds((ci + 1) * L, L)] = wpair

    def wait_rows(pj):
      pltpu.make_async_copy(
          fm_hbm.at[pl.ds(0, 4 * CH)], rows[pj], rsem[pj]).wait()

    def combine_and_fire(j, pj):
      c = chunk_of(j)
      n0 = c * CH
      for p in range(CH):
        wb = [plsc.bitcast(plsc.load_gather(
            wv[pj], [jnp.full((L,), (ci + 1) * L + p, jnp.int32)]),
            jnp.bfloat16) for ci in range(4)]
        for r in range(C // (2 * L)):
          sl = pl.ds(r * 2 * L, 2 * L)
          acc = wb[0] * rows[pj][p, sl]
          acc = acc + wb[1] * rows[pj][CH + p, sl]
          acc = acc + wb[2] * rows[pj][2 * CH + p, sl]
          acc = acc + wb[3] * rows[pj][3 * CH + p, sl]
          outv[pj][p, sl] = acc
      pltpu.async_copy(outv[pj], out_hbm.at[pl.ds(b * N + n0, CH)], osem[pj])

    def wait_out(pj):
      pltpu.make_async_copy(
          out_hbm.at[pl.ds(0, CH)], outv[pj], osem[pj]).wait()

    fire_parents(0, 0)
    wait_parents(0)
    compute_and_fire(0, 0)
    fire_parents(1, 1)

    def phase(jj, j, pj):
      nxt = 1 - pj

      @pl.when(chunk_of(j + 1) < NCHUNK)
      def _():
        wait_parents(nxt)
        compute_and_fire(j + 1, nxt)

      @pl.when(chunk_of(j + 2) < NCHUNK)
      def _():
        fire_parents(j + 2, pj)

      @pl.when(chunk_of(j) < NCHUNK)
      def _():
        wait_rows(pj)

        @pl.when(jj >= 1)
        def _():
          wait_out(pj)

        combine_and_fire(j, pj)

    def body(jj, carry):
      phase(jj, 2 * jj, 0)
      phase(jj, 2 * jj + 1, 1)
      return carry

    lax.fori_loop(0, NPH // 2, body, 0)
    wait_out(0)
    wait_out(1)

  return k


@jax.jit
def kernel(feature_map, vertices2d, parents, bary):
  B, C, H, W = feature_map.shape
  N = parents.shape[0]
  Nv = vertices2d.shape[1]
  K = bary.shape[0]
  fm_t = jnp.transpose(feature_map, (0, 2, 3, 1)).reshape(B * H * W, C)
  fm_bf = fm_t.astype(jnp.bfloat16)
  pad = (-Nv * 2) % 8
  verts = jnp.pad(vertices2d.reshape(B, Nv * 2), ((0, 0), (0, pad))).reshape(-1)
  k = _sc_kernel(B, C, H, W, N, Nv, K)
  out = k(fm_bf, verts, parents.reshape(N * 3), bary.reshape(K * 3))
  return out.astype(jnp.float32).reshape(B, N, C)


# speedup vs baseline: 1.4645x; 1.4645x over previous
import functools

import jax
import jax.numpy as jnp
from jax import lax
from jax.experimental import pallas as pl
from jax.experimental.pallas import tpu as pltpu, tpu_sc as plsc

NC, NS, L = 2, 16, 16
NW = NC * NS
TPB = NS // 2
CH = 16


def _sc_kernel(B, C, H, W, N, Nv, K):
  HW = H * W
  NCHUNK = N // CH
  NV2 = Nv * 2 + (-Nv * 2) % 8
  NPH = 2 * ((NCHUNK // TPB + 2) // 2)
  mesh = plsc.VectorSubcoreMesh(
      core_axis_name="c", subcore_axis_name="s", num_cores=NC, num_subcores=NS)

  @functools.partial(
      pl.kernel,
      out_type=jax.ShapeDtypeStruct((B * N, C), jnp.float32),
      mesh=mesh,
      compiler_params=pltpu.CompilerParams(
          needs_layout_passes=False, use_tc_tiling_on_sc=False),
      scratch_types=[
          pltpu.VMEM((NV2,), jnp.float32),
          pltpu.VMEM((K * 3,), jnp.float32),
          pltpu.VMEM((CH * 3,), jnp.int32),
          pltpu.VMEM((CH * 3,), jnp.int32),
          pltpu.VMEM((5 * CH,), jnp.int32),
          pltpu.VMEM((5 * CH,), jnp.int32),
          pltpu.VMEM((4 * CH, C), jnp.bfloat16),
          pltpu.VMEM((4 * CH, C), jnp.bfloat16),
          pltpu.VMEM((CH, C), jnp.float32),
          pltpu.VMEM((CH, C), jnp.float32),
          pltpu.SemaphoreType.DMA,
          pltpu.SemaphoreType.DMA,
          pltpu.SemaphoreType.DMA,
          pltpu.SemaphoreType.DMA,
          pltpu.SemaphoreType.DMA,
          pltpu.SemaphoreType.DMA,
      ],
  )
  def k(fm_hbm, verts_hbm, parents_hbm, bary_hbm, out_hbm,
        verts_v, bary_v, pv0, pv1, wv0, wv1, rows0, rows1, outv0, outv1,
        psem0, psem1, rsem0, rsem1, osem0, osem1):
    wid = lax.axis_index("s") * NC + lax.axis_index("c")
    b = wid // TPB
    s = wid % TPB
    pv = (pv0, pv1)
    wv = (wv0, wv1)
    rows = (rows0, rows1)
    outv = (outv0, outv1)
    psem = (psem0, psem1)
    rsem = (rsem0, rsem1)
    osem = (osem0, osem1)

    pltpu.sync_copy(verts_hbm.at[pl.ds(b * NV2, NV2)], verts_v)
    pltpu.sync_copy(bary_hbm, bary_v)

    lane = lax.broadcasted_iota(jnp.int32, (L,), 0)
    lane3 = lane * 3
    fmbase = b * HW

    def chunk_of(j):
      return s + TPB * j

    def fire_parents(j, pj):
      c = chunk_of(j)
      pltpu.async_copy(
          parents_hbm.at[pl.ds(c * (CH * 3), CH * 3)], pv[pj], psem[pj])

    def wait_parents(pj):
      pltpu.make_async_copy(
          parents_hbm.at[pl.ds(0, CH * 3)], pv[pj], psem[pj]).wait()

    def compute_and_fire(j, pj):
      c = chunk_of(j)
      n0 = c * CH
      nvec = n0 + lane
      bidx = lax.rem(nvec, K) * 3
      b0 = plsc.load_gather(bary_v, [bidx])
      b1 = plsc.load_gather(bary_v, [bidx + 1])
      b2 = plsc.load_gather(bary_v, [bidx + 2])
      p0 = plsc.load_gather(pv[pj], [lane3]) * 2
      p1 = plsc.load_gather(pv[pj], [lane3 + 1]) * 2
      p2 = plsc.load_gather(pv[pj], [lane3 + 2]) * 2
      vx = (b0 * plsc.load_gather(verts_v, [p0])
            + b1 * plsc.load_gather(verts_v, [p1])
            + b2 * plsc.load_gather(verts_v, [p2]))
      vy = (b0 * plsc.load_gather(verts_v, [p0 + 1])
            + b1 * plsc.load_gather(verts_v, [p1 + 1])
            + b2 * plsc.load_gather(verts_v, [p2 + 1]))
      x = vx / (W - 1.0) * 2.0 - 1.0
      y = vy / (H - 1.0) * 2.0 - 1.0
      ix = (x + 1.0) * 0.5 * (W - 1.0)
      iy = (y + 1.0) * 0.5 * (H - 1.0)
      txi = ix.astype(jnp.int32)
      ix0 = jnp.where(ix < txi.astype(jnp.float32), txi - 1, txi)
      tyi = iy.astype(jnp.int32)
      iy0 = jnp.where(iy < tyi.astype(jnp.float32), tyi - 1, tyi)
      fx = ix - ix0.astype(jnp.float32)
      fy = iy - iy0.astype(jnp.float32)
      wx = (1.0 - fx, fx)
      wy = (1.0 - fy, fy)
      for ci, (dx, dy) in enumerate(((0, 0), (1, 0), (0, 1), (1, 1))):
        xi = ix0 + dx
        yi = iy0 + dy
        valid = ((xi >= 0) & (xi <= W - 1) & (yi >= 0) & (yi <= H - 1))
        xc = jnp.clip(xi, 0, W - 1)
        yc = jnp.clip(yi, 0, H - 1)
        lin = fmbase + yc * W + xc
        pltpu.async_copy(
            fm_hbm.at[lin], rows[pj].at[pl.ds(ci * CH, CH)], rsem[pj])
        w = jnp.where(valid, wx[dx] * wy[dy], 0.0)
        wpair = plsc.bitcast(
            plsc.pack(w, w, format=plsc.PackFormat.INTERLEAVED), jnp.int32)
        wv[pj][pl.ds((ci + 1) * L, L)] = wpair

    def wait_rows(pj):
      pltpu.make_async_copy(
          fm_hbm.at[pl.ds(0, 4 * CH)], rows[pj], rsem[pj]).wait()

    def combine_and_fire(j, pj):
      c = chunk_of(j)
      n0 = c * CH
      for p in range(CH):
        wb = [plsc.bitcast(plsc.load_gather(
            wv[pj], [jnp.full((L,), (ci + 1) * L + p, jnp.int32)]),
            jnp.bfloat16) for ci in range(4)]
        for r in range(C // (2 * L)):
          sl = pl.ds(r * 2 * L, 2 * L)
          acc = wb[0] * rows[pj][p, sl]
          acc = acc + wb[1] * rows[pj][CH + p, sl]
          acc = acc + wb[2] * rows[pj][2 * CH + p, sl]
          acc = acc + wb[3] * rows[pj][3 * CH + p, sl]
          va, vb = plsc.unpack(acc, format=plsc.PackFormat.INTERLEAVED)
          outv[pj][p, pl.ds(r * 2 * L, L)] = va
          outv[pj][p, pl.ds(r * 2 * L + L, L)] = vb
      pltpu.async_copy(outv[pj], out_hbm.at[pl.ds(b * N + n0, CH)], osem[pj])

    def wait_out(pj):
      pltpu.make_async_copy(
          out_hbm.at[pl.ds(0, CH)], outv[pj], osem[pj]).wait()

    fire_parents(0, 0)
    wait_parents(0)
    compute_and_fire(0, 0)
    fire_parents(1, 1)

    def phase(jj, j, pj):
      nxt = 1 - pj

      @pl.when(chunk_of(j + 1) < NCHUNK)
      def _():
        wait_parents(nxt)
        compute_and_fire(j + 1, nxt)

      @pl.when(chunk_of(j + 2) < NCHUNK)
      def _():
        fire_parents(j + 2, pj)

      @pl.when(chunk_of(j) < NCHUNK)
      def _():
        wait_rows(pj)

        @pl.when(jj >= 1)
        def _():
          wait_out(pj)

        combine_and_fire(j, pj)

    def body(jj, carry):
      phase(jj, 2 * jj, 0)
      phase(jj, 2 * jj + 1, 1)
      return carry

    lax.fori_loop(0, NPH // 2, body, 0)
    wait_out(0)
    wait_out(1)

  return k


@jax.jit
def kernel(feature_map, vertices2d, parents, bary):
  B, C, H, W = feature_map.shape
  N = parents.shape[0]
  Nv = vertices2d.shape[1]
  K = bary.shape[0]
  fm_t = jnp.transpose(feature_map, (0, 2, 3, 1)).reshape(B * H * W, C)
  perm = (32 * jnp.arange(C // 32)[:, None, None]
          + jnp.arange(16)[None, :, None]
          + 16 * jnp.arange(2)[None, None, :]).reshape(C)
  fm_bf = fm_t.astype(jnp.bfloat16)[:, perm]
  pad = (-Nv * 2) % 8
  verts = jnp.pad(vertices2d.reshape(B, Nv * 2), ((0, 0), (0, pad))).reshape(-1)
  k = _sc_kernel(B, C, H, W, N, Nv, K)
  out = k(fm_bf, verts, parents.reshape(N * 3), bary.reshape(K * 3))
  return out.reshape(B, N, C)


# speedup vs baseline: 1.5668x; 1.0698x over previous
import functools

import jax
import jax.numpy as jnp
from jax import lax
from jax.experimental import pallas as pl
from jax.experimental.pallas import tpu as pltpu, tpu_sc as plsc

NC, NS, L = 2, 16, 16
NW = NC * NS
TPB = NS // 2
CH = 16


def _sc_kernel(B, C, H, W, N, Nv, K):
  HW = H * W
  NCHUNK = N // CH
  NV2 = Nv * 2 + (-Nv * 2) % 8
  NPH = 2 * ((NCHUNK // TPB + 2) // 2)
  mesh = plsc.VectorSubcoreMesh(
      core_axis_name="c", subcore_axis_name="s", num_cores=NC, num_subcores=NS)

  @functools.partial(
      pl.kernel,
      out_type=jax.ShapeDtypeStruct((B * N, C), jnp.float32),
      mesh=mesh,
      compiler_params=pltpu.CompilerParams(
          needs_layout_passes=False, use_tc_tiling_on_sc=False),
      scratch_types=[
          pltpu.VMEM((NV2,), jnp.float32),
          pltpu.VMEM((K * 3,), jnp.float32),
          pltpu.VMEM((CH * 3,), jnp.int32),
          pltpu.VMEM((CH * 3,), jnp.int32),
          pltpu.VMEM((5 * CH,), jnp.float32),
          pltpu.VMEM((5 * CH,), jnp.float32),
          pltpu.VMEM((4 * CH, C), jnp.float32),
          pltpu.VMEM((4 * CH, C), jnp.float32),
          pltpu.VMEM((CH, C), jnp.float32),
          pltpu.VMEM((CH, C), jnp.float32),
          pltpu.SemaphoreType.DMA,
          pltpu.SemaphoreType.DMA,
          pltpu.SemaphoreType.DMA,
          pltpu.SemaphoreType.DMA,
          pltpu.SemaphoreType.DMA,
          pltpu.SemaphoreType.DMA,
      ],
  )
  def k(fm_hbm, verts_hbm, parents_hbm, bary_hbm, out_hbm,
        verts_v, bary_v, pv0, pv1, wv0, wv1, rows0, rows1, outv0, outv1,
        psem0, psem1, rsem0, rsem1, osem0, osem1):
    wid = lax.axis_index("s") * NC + lax.axis_index("c")
    b = wid // TPB
    s = wid % TPB
    pv = (pv0, pv1)
    wv = (wv0, wv1)
    rows = (rows0, rows1)
    outv = (outv0, outv1)
    psem = (psem0, psem1)
    rsem = (rsem0, rsem1)
    osem = (osem0, osem1)

    pltpu.sync_copy(verts_hbm.at[pl.ds(b * NV2, NV2)], verts_v)
    pltpu.sync_copy(bary_hbm, bary_v)

    lane = lax.broadcasted_iota(jnp.int32, (L,), 0)
    lane3 = lane * 3
    fmbase = b * HW

    def chunk_of(j):
      return s + TPB * j

    def fire_parents(j, pj):
      c = chunk_of(j)
      pltpu.async_copy(
          parents_hbm.at[pl.ds(c * (CH * 3), CH * 3)], pv[pj], psem[pj])

    def wait_parents(pj):
      pltpu.make_async_copy(
          parents_hbm.at[pl.ds(0, CH * 3)], pv[pj], psem[pj]).wait()

    def compute_and_fire(j, pj):
      c = chunk_of(j)
      n0 = c * CH
      nvec = n0 + lane
      bidx = lax.rem(nvec, K) * 3
      b0 = plsc.load_gather(bary_v, [bidx])
      b1 = plsc.load_gather(bary_v, [bidx + 1])
      b2 = plsc.load_gather(bary_v, [bidx + 2])
      p0 = plsc.load_gather(pv[pj], [lane3]) * 2
      p1 = plsc.load_gather(pv[pj], [lane3 + 1]) * 2
      p2 = plsc.load_gather(pv[pj], [lane3 + 2]) * 2
      vx = (b0 * plsc.load_gather(verts_v, [p0])
            + b1 * plsc.load_gather(verts_v, [p1])
            + b2 * plsc.load_gather(verts_v, [p2]))
      vy = (b0 * plsc.load_gather(verts_v, [p0 + 1])
            + b1 * plsc.load_gather(verts_v, [p1 + 1])
            + b2 * plsc.load_gather(verts_v, [p2 + 1]))
      x = vx / (W - 1.0) * 2.0 - 1.0
      y = vy / (H - 1.0) * 2.0 - 1.0
      ix = (x + 1.0) * 0.5 * (W - 1.0)
      iy = (y + 1.0) * 0.5 * (H - 1.0)
      txi = ix.astype(jnp.int32)
      ix0 = jnp.where(ix < txi.astype(jnp.float32), txi - 1, txi)
      tyi = iy.astype(jnp.int32)
      iy0 = jnp.where(iy < tyi.astype(jnp.float32), tyi - 1, tyi)
      fx = ix - ix0.astype(jnp.float32)
      fy = iy - iy0.astype(jnp.float32)
      wx = (1.0 - fx, fx)
      wy = (1.0 - fy, fy)
      for ci, (dx, dy) in enumerate(((0, 0), (1, 0), (0, 1), (1, 1))):
        xi = ix0 + dx
        yi = iy0 + dy
        valid = ((xi >= 0) & (xi <= W - 1) & (yi >= 0) & (yi <= H - 1))
        xc = jnp.clip(xi, 0, W - 1)
        yc = jnp.clip(yi, 0, H - 1)
        lin = fmbase + yc * W + xc
        pltpu.async_copy(
            fm_hbm.at[lin], rows[pj].at[pl.ds(ci * CH, CH)], rsem[pj])
        wv[pj][pl.ds((ci + 1) * L, L)] = jnp.where(valid, wx[dx] * wy[dy], 0.0)

    def wait_rows(pj):
      pltpu.make_async_copy(
          fm_hbm.at[pl.ds(0, 4 * CH)], rows[pj], rsem[pj]).wait()

    def combine_and_fire(j, pj):
      c = chunk_of(j)
      n0 = c * CH
      for p in range(CH):
        wb = [plsc.load_gather(
            wv[pj], [jnp.full((L,), (ci + 1) * L + p, jnp.int32)])
            for ci in range(4)]
        for r in range(C // L):
          sl = pl.ds(r * L, L)
          acc = wb[0] * rows[pj][p, sl]
          acc = acc + wb[1] * rows[pj][CH + p, sl]
          acc = acc + wb[2] * rows[pj][2 * CH + p, sl]
          acc = acc + wb[3] * rows[pj][3 * CH + p, sl]
          outv[pj][p, sl] = acc
      pltpu.async_copy(outv[pj], out_hbm.at[pl.ds(b * N + n0, CH)], osem[pj])

    def wait_out(pj):
      pltpu.make_async_copy(
          out_hbm.at[pl.ds(0, CH)], outv[pj], osem[pj]).wait()

    fire_parents(0, 0)
    wait_parents(0)
    compute_and_fire(0, 0)
    fire_parents(1, 1)

    def phase(jj, j, pj):
      nxt = 1 - pj

      @pl.when(chunk_of(j + 1) < NCHUNK)
      def _():
        wait_parents(nxt)
        compute_and_fire(j + 1, nxt)

      @pl.when(chunk_of(j + 2) < NCHUNK)
      def _():
        fire_parents(j + 2, pj)

      @pl.when(chunk_of(j) < NCHUNK)
      def _():
        wait_rows(pj)

        @pl.when(jj >= 1)
        def _():
          wait_out(pj)

        combine_and_fire(j, pj)

    def body(jj, carry):
      phase(jj, 2 * jj, 0)
      phase(jj, 2 * jj + 1, 1)
      return carry

    lax.fori_loop(0, NPH // 2, body, 0)
    wait_out(0)
    wait_out(1)

  return k


@jax.jit
def kernel(feature_map, vertices2d, parents, bary):
  B, C, H, W = feature_map.shape
  N = parents.shape[0]
  Nv = vertices2d.shape[1]
  K = bary.shape[0]
  fm_t = jnp.transpose(feature_map, (0, 2, 3, 1)).reshape(B * H * W, C)
  pad = (-Nv * 2) % 8
  verts = jnp.pad(vertices2d.reshape(B, Nv * 2), ((0, 0), (0, pad))).reshape(-1)
  k = _sc_kernel(B, C, H, W, N, Nv, K)
  out = k(fm_t, verts, parents.reshape(N * 3), bary.reshape(K * 3))
  return out.reshape(B, N, C)
